# Initial kernel scaffold; baseline (speedup 1.0000x reference)
import jax
import jax.numpy as jnp
from jax.experimental import pallas as pl


def kernel(x, table):
    raise NotImplementedError("write your pallas kernel here")


# speedup vs baseline: 6.8448x; 6.8448x over previous
import functools

import jax
import jax.numpy as jnp
from jax import lax
from jax.experimental import pallas as pl
from jax.experimental.pallas import tpu as pltpu
from jax.experimental.pallas import tpu_sc as plsc

B, L, TAGS, D = 4096, 50, 20, 16
NUM_SEG = B * L
NC, NS = 2, 16
NW = NC * NS
SEG_PER_W = NUM_SEG // NW
C = 128
CHUNKS = SEG_PER_W // C
G = C * TAGS // 128


def _sc_segment_sum(x_flat, table):
    mesh = plsc.VectorSubcoreMesh(core_axis_name="c", subcore_axis_name="s")

    @functools.partial(
        pl.kernel,
        mesh=mesh,
        out_type=jax.ShapeDtypeStruct((NUM_SEG, D), jnp.float32),
        scratch_types=[
            pltpu.VMEM((C * TAGS,), jnp.int32),
            pltpu.VMEM((C * TAGS, D), jnp.float32),
            pltpu.VMEM((C, D), jnp.float32),
            pltpu.SemaphoreType.DMA,
        ],
        compiler_params=pltpu.CompilerParams(use_tc_tiling_on_sc=False),
    )
    def sc_kernel(idx_hbm, table_hbm, out_hbm, idx_v, rows_v, out_v, sem):
        wid = lax.axis_index("s") * NC + lax.axis_index("c")

        def chunk_body(ci, _):
            seg_base = wid * SEG_PER_W + ci * C
            pltpu.sync_copy(idx_hbm.at[pl.ds(seg_base * TAGS, C * TAGS)], idx_v)
            copies = []
            for j in range(G):
                copies.append(
                    pltpu.async_copy(
                        table_hbm.at[idx_v.at[pl.ds(j * 128, 128)]],
                        rows_v.at[pl.ds(j * 128, 128)],
                        sem,
                    )
                )
            for cp in copies:
                cp.wait()

            def seg_body(s, _):
                base = s * TAGS
                acc = rows_v[base]
                for t in range(1, TAGS):
                    acc = acc + rows_v[base + t]
                out_v[s] = acc
                return 0

            lax.fori_loop(0, C, seg_body, 0)
            pltpu.sync_copy(out_v, out_hbm.at[pl.ds(seg_base, C)])
            return 0

        lax.fori_loop(0, CHUNKS, chunk_body, 0)

    return sc_kernel(x_flat, table)


def _tc_divide(x2d, sums):
    blk = 2048

    def body(x_ref, s_ref, o_ref):
        cnt = jnp.sum((x_ref[...] > 0).astype(jnp.float32), axis=-1,
                      keepdims=True)
        o_ref[...] = s_ref[...] / (cnt + 1e-6)

    return pl.pallas_call(
        body,
        grid=(NUM_SEG // blk,),
        in_specs=[
            pl.BlockSpec((blk, TAGS), lambda i: (i, 0)),
            pl.BlockSpec((blk, D), lambda i: (i, 0)),
        ],
        out_specs=pl.BlockSpec((blk, D), lambda i: (i, 0)),
        out_shape=jax.ShapeDtypeStruct((NUM_SEG, D), jnp.float32),
    )(x2d, sums)


def kernel(x, table):
    x_flat = x.reshape(-1)
    sums = _sc_segment_sum(x_flat, table)
    out = _tc_divide(x.reshape(NUM_SEG, TAGS), sums)
    return out.reshape(B, L, D)


# speedup vs baseline: 7.8445x; 1.1461x over previous
import functools

import jax
import jax.numpy as jnp
from jax import lax
from jax.experimental import pallas as pl
from jax.experimental.pallas import tpu as pltpu
from jax.experimental.pallas import tpu_sc as plsc

B, L, TAGS, D = 4096, 50, 20, 16
NUM_SEG = B * L
NC, NS = 2, 16
NW = NC * NS
SEG_PER_W = NUM_SEG // NW
C = 128
CHUNKS = SEG_PER_W // C
G = C * TAGS // 128


def _sc_segment_sum(x_flat, table):
    mesh = plsc.VectorSubcoreMesh(core_axis_name="c", subcore_axis_name="s")

    @functools.partial(
        pl.kernel,
        mesh=mesh,
        out_type=jax.ShapeDtypeStruct((NUM_SEG, D), jnp.float32),
        scratch_types=[
            pltpu.VMEM((C * TAGS,), jnp.int32),
            pltpu.VMEM((C * TAGS,), jnp.int32),
            pltpu.VMEM((C * TAGS, D), jnp.float32),
            pltpu.VMEM((C * TAGS, D), jnp.float32),
            pltpu.VMEM((C, D), jnp.float32),
            pltpu.VMEM((C, D), jnp.float32),
            pltpu.SemaphoreType.DMA,
            pltpu.SemaphoreType.DMA,
        ],
        compiler_params=pltpu.CompilerParams(use_tc_tiling_on_sc=False),
    )
    def sc_kernel(idx_hbm, table_hbm, out_hbm, idx_v0, idx_v1, rows_v0,
                  rows_v1, out_v0, out_v1, gsem0, gsem1):
        idx_vs = (idx_v0, idx_v1)
        rows_vs = (rows_v0, rows_v1)
        out_vs = (out_v0, out_v1)
        gsems = (gsem0, gsem1)
        wid = lax.axis_index("s") * NC + lax.axis_index("c")
        wbase = wid * SEG_PER_W

        def fire(ci, b):
            seg_base = wbase + ci * C
            pltpu.sync_copy(idx_hbm.at[pl.ds(seg_base * TAGS, C * TAGS)],
                            idx_vs[b])
            for j in range(G):
                pltpu.async_copy(
                    table_hbm.at[idx_vs[b].at[pl.ds(j * 128, 128)]],
                    rows_vs[b].at[pl.ds(j * 128, 128)],
                    gsems[b],
                )

        def drain(b):
            pltpu.make_async_copy(
                table_hbm.at[pl.ds(0, C * TAGS)], rows_vs[b], gsems[b]
            ).wait()

        def compute(ci, b):
            seg_base = wbase + ci * C
            rows = rows_vs[b]
            out_v = out_vs[b]

            def seg_body(s, _):
                base = s * TAGS
                acc = rows[base]
                for t in range(1, TAGS):
                    acc = acc + rows[base + t]
                out_v[s] = acc
                return 0

            lax.fori_loop(0, C, seg_body, 0)
            pltpu.sync_copy(out_v, out_hbm.at[pl.ds(seg_base, C)])

        fire(0, 0)

        def pair_body(p, _):
            ci = 2 * p
            fire(ci + 1, 1)
            drain(0)
            compute(ci, 0)

            @pl.when(ci + 2 < CHUNKS)
            def _():
                fire(ci + 2, 0)

            drain(1)
            compute(ci + 1, 1)
            return 0

        lax.fori_loop(0, CHUNKS // 2, pair_body, 0)

    return sc_kernel(x_flat, table)


def _tc_divide(x2d, sums):
    blk = 2048

    def body(x_ref, s_ref, o_ref):
        cnt = jnp.sum((x_ref[...] > 0).astype(jnp.float32), axis=-1,
                      keepdims=True)
        o_ref[...] = s_ref[...] / (cnt + 1e-6)

    return pl.pallas_call(
        body,
        grid=(NUM_SEG // blk,),
        in_specs=[
            pl.BlockSpec((blk, TAGS), lambda i: (i, 0)),
            pl.BlockSpec((blk, D), lambda i: (i, 0)),
        ],
        out_specs=pl.BlockSpec((blk, D), lambda i: (i, 0)),
        out_shape=jax.ShapeDtypeStruct((NUM_SEG, D), jnp.float32),
    )(x2d, sums)


def kernel(x, table):
    x_flat = x.reshape(-1)
    sums = _sc_segment_sum(x_flat, table)
    out = _tc_divide(x.reshape(NUM_SEG, TAGS), sums)
    return out.reshape(B, L, D)


# speedup vs baseline: 9.1849x; 1.1709x over previous
import functools

import jax
import jax.numpy as jnp
from jax import lax
from jax.experimental import pallas as pl
from jax.experimental.pallas import tpu as pltpu
from jax.experimental.pallas import tpu_sc as plsc

B, L, TAGS, D = 4096, 50, 20, 16
NUM_SEG = B * L
NC, NS = 2, 16
NW = NC * NS
SEG_PER_W = NUM_SEG // NW
C = 128
CHUNKS = SEG_PER_W // C
G = C * TAGS // 128


def _sc_segment_sum(x_flat, table):
    mesh = plsc.VectorSubcoreMesh(core_axis_name="c", subcore_axis_name="s")

    @functools.partial(
        pl.kernel,
        mesh=mesh,
        out_type=jax.ShapeDtypeStruct((NUM_SEG, D), jnp.float32),
        scratch_types=[
            pltpu.VMEM((C * TAGS,), jnp.int32),
            pltpu.VMEM((C * TAGS,), jnp.int32),
            pltpu.VMEM((C * TAGS, D), jnp.float32),
            pltpu.VMEM((C * TAGS, D), jnp.float32),
            pltpu.VMEM((C, D), jnp.float32),
            pltpu.VMEM((C, D), jnp.float32),
            pltpu.SemaphoreType.DMA,
            pltpu.SemaphoreType.DMA,
        ],
        compiler_params=pltpu.CompilerParams(use_tc_tiling_on_sc=False,
                                             needs_layout_passes=False),
    )
    def sc_kernel(idx_hbm, table_hbm, out_hbm, idx_v0, idx_v1, rows_v0,
                  rows_v1, out_v0, out_v1, gsem0, gsem1):
        idx_vs = (idx_v0, idx_v1)
        rows_vs = (rows_v0, rows_v1)
        out_vs = (out_v0, out_v1)
        gsems = (gsem0, gsem1)
        wid = lax.axis_index("s") * NC + lax.axis_index("c")
        wbase = wid * SEG_PER_W

        def fire(ci, b):
            seg_base = wbase + ci * C
            pltpu.sync_copy(idx_hbm.at[pl.ds(seg_base * TAGS, C * TAGS)],
                            idx_vs[b])
            for j in range(G):
                pltpu.async_copy(
                    table_hbm.at[idx_vs[b].at[pl.ds(j * 128, 128)]],
                    rows_vs[b].at[pl.ds(j * 128, 128)],
                    gsems[b],
                )

        def drain(b):
            pltpu.make_async_copy(
                table_hbm.at[pl.ds(0, C * TAGS)], rows_vs[b], gsems[b]
            ).wait()

        def compute(ci, b):
            seg_base = wbase + ci * C
            rows = rows_vs[b]
            idx_v = idx_vs[b]
            out_v = out_vs[b]

            lane = lax.iota(jnp.int32, 16) * TAGS

            def grp_body(q, _):
                tag0 = lane + q * (16 * TAGS)
                cnt = jnp.zeros((16,), jnp.float32)
                for t in range(TAGS):
                    vals = plsc.load_gather(idx_v, [tag0 + t])
                    cnt = cnt + (vals > 0).astype(jnp.float32)
                rv = 1.0 / (cnt + 1e-6)
                for r in range(16):
                    base = (q * 16 + r) * TAGS
                    acc = rows[base]
                    for t in range(1, TAGS):
                        acc = acc + rows[base + t]
                    out_v[q * 16 + r] = acc * rv[r]
                return 0

            lax.fori_loop(0, C // 16, grp_body, 0)
            pltpu.sync_copy(out_v, out_hbm.at[pl.ds(seg_base, C)])

        fire(0, 0)

        def pair_body(p, _):
            ci = 2 * p
            fire(ci + 1, 1)
            drain(0)
            compute(ci, 0)

            @pl.when(ci + 2 < CHUNKS)
            def _():
                fire(ci + 2, 0)

            drain(1)
            compute(ci + 1, 1)
            return 0

        lax.fori_loop(0, CHUNKS // 2, pair_body, 0)

    return sc_kernel(x_flat, table)


def kernel(x, table):
    x_flat = x.reshape(-1)
    out = _sc_segment_sum(x_flat, table)
    return out.reshape(B, L, D)
